# Initial kernel scaffold; baseline (speedup 1.0000x reference)
import functools

import jax
import jax.numpy as jnp
from jax import lax
from jax.experimental import pallas as pl
from jax.experimental.pallas import tpu as pltpu
from jax.experimental.pallas import tpu_sc as plsc

N_NODES = 10000
D = 128
NC = 2
NS = 16
L = 16
NW = NC * NS
CHUNK = 128
N_ROWS = 10240
PAD = N_NODES
ROWS_PER_SUB = N_ROWS // NS
N_EDGES = 320000
E_PAD = ((N_EDGES + NW * CHUNK - 1) // (NW * CHUNK)) * (NW * CHUNK)
CPT = E_PAD // (NW * CHUNK)
IDS_PAD = 12288
IDS_CPT = IDS_PAD // (NW * CHUNK)

_mesh = plsc.VectorSubcoreMesh(core_axis_name="c", subcore_axis_name="s")


@functools.partial(
    pl.kernel,
    out_type=[
        jax.ShapeDtypeStruct((IDS_PAD, D), jnp.float32),
        jax.ShapeDtypeStruct((NC, N_ROWS, L), jnp.float32),
        jax.ShapeDtypeStruct((NC, N_ROWS, L), jnp.float32),
    ],
    mesh=_mesh,
    scratch_types=[
        pltpu.VMEM((CHUNK,), jnp.int32),
        pltpu.VMEM((CHUNK, L), jnp.float32),
        pltpu.VMEM((ROWS_PER_SUB, L), jnp.float32),
        pltpu.VMEM((CHUNK, D), jnp.float32),
        pltpu.VMEM_SHARED((N_ROWS, L), jnp.float32),
        pltpu.VMEM_SHARED((N_ROWS, L), jnp.float32),
    ],
)
def _sc_prep(ids_hbm, src_hbm, dst_hbm, emb_hbm,
             h0_hbm, hsrc_hbm, hdst_hbm,
             idx_v, ones_v, z16_v, rows_v, hist_src_sh, hist_dst_sh):
    c = lax.axis_index("c")
    s = lax.axis_index("s")
    wid = s * NC + c

    @pl.loop(0, CHUNK)
    def _(i):
        ones_v[i, :] = jnp.full((L,), 1.0, jnp.float32)

    @pl.loop(0, ROWS_PER_SUB)
    def _(i):
        z16_v[i, :] = jnp.zeros((L,), jnp.float32)

    pltpu.sync_copy(z16_v, hist_src_sh.at[pl.ds(s * ROWS_PER_SUB, ROWS_PER_SUB)])
    pltpu.sync_copy(z16_v, hist_dst_sh.at[pl.ds(s * ROWS_PER_SUB, ROWS_PER_SUB)])
    plsc.subcore_barrier()

    @pl.loop(0, CPT)
    def _(j):
        base = (wid * CPT + j) * CHUNK
        pltpu.sync_copy(src_hbm.at[pl.ds(base, CHUNK)], idx_v)
        pltpu.sync_copy(ones_v, hist_src_sh.at[idx_v], add=True)
        pltpu.sync_copy(dst_hbm.at[pl.ds(base, CHUNK)], idx_v)
        pltpu.sync_copy(ones_v, hist_dst_sh.at[idx_v], add=True)

    @pl.loop(0, IDS_CPT)
    def _(j):
        base = (wid * IDS_CPT + j) * CHUNK
        pltpu.sync_copy(ids_hbm.at[pl.ds(base, CHUNK)], idx_v)
        pltpu.sync_copy(emb_hbm.at[idx_v], rows_v)
        pltpu.sync_copy(rows_v, h0_hbm.at[pl.ds(base, CHUNK)])

    plsc.subcore_barrier()
    pltpu.sync_copy(hist_src_sh.at[pl.ds(s * ROWS_PER_SUB, ROWS_PER_SUB)],
                    hsrc_hbm.at[c, pl.ds(s * ROWS_PER_SUB, ROWS_PER_SUB)])
    pltpu.sync_copy(hist_dst_sh.at[pl.ds(s * ROWS_PER_SUB, ROWS_PER_SUB)],
                    hdst_hbm.at[c, pl.ds(s * ROWS_PER_SUB, ROWS_PER_SUB)])


@functools.partial(
    pl.kernel,
    out_type=jax.ShapeDtypeStruct((NC, N_ROWS, D), jnp.float32),
    mesh=_mesh,
    scratch_types=[
        pltpu.VMEM((CHUNK,), jnp.int32),
        pltpu.VMEM((CHUNK,), jnp.int32),
        pltpu.VMEM((CHUNK, D), jnp.float32),
        pltpu.VMEM((CHUNK, D), jnp.float32),
        pltpu.VMEM_SHARED((N_ROWS, D), jnp.float32),
    ],
)
def _sc_agg(h_hbm, src_hbm, dst_hbm, out_hbm,
            sidx_v, didx_v, rows_v, zrow_v, agg_sh):
    c = lax.axis_index("c")
    s = lax.axis_index("s")
    wid = s * NC + c

    @pl.loop(0, CHUNK)
    def _(i):
        @pl.loop(0, D // L)
        def _(k):
            zrow_v[i, pl.ds(k * L, L)] = jnp.zeros((L,), jnp.float32)

    @pl.loop(0, ROWS_PER_SUB // CHUNK)
    def _(r):
        pltpu.sync_copy(zrow_v, agg_sh.at[pl.ds(s * ROWS_PER_SUB + r * CHUNK, CHUNK)])
    plsc.subcore_barrier()

    @pl.loop(0, CPT)
    def _(j):
        base = (wid * CPT + j) * CHUNK
        pltpu.sync_copy(src_hbm.at[pl.ds(base, CHUNK)], sidx_v)
        pltpu.sync_copy(dst_hbm.at[pl.ds(base, CHUNK)], didx_v)
        pltpu.sync_copy(h_hbm.at[sidx_v], rows_v)
        pltpu.sync_copy(rows_v, agg_sh.at[didx_v], add=True)

    plsc.subcore_barrier()
    pltpu.sync_copy(agg_sh.at[pl.ds(s * ROWS_PER_SUB, ROWS_PER_SUB)],
                    out_hbm.at[c, pl.ds(s * ROWS_PER_SUB, ROWS_PER_SUB)])


def _norm_from_hist(h_ref):
    deg = h_ref[0, :, 0:1] + h_ref[1, :, 0:1]
    return lax.rsqrt(jnp.maximum(deg, 1.0))


def _tc_scale0_body(h0_ref, hs_ref, o_ref):
    o_ref[...] = h0_ref[:N_ROWS, :] * _norm_from_hist(hs_ref)


def _tc_layer1_body(p_ref, hd_ref, hs_ref, w_ref, b_ref, o_ref):
    agg = (p_ref[0] + p_ref[1]) * _norm_from_hist(hd_ref)
    y = jnp.dot(agg, w_ref[...], preferred_element_type=jnp.float32)
    y = y + b_ref[...][None, :]
    o_ref[...] = jnp.maximum(y, 0.0) * _norm_from_hist(hs_ref)


def _tc_layer2_body(p_ref, hd_ref, w_ref, b_ref, o_ref):
    agg = (p_ref[0] + p_ref[1]) * _norm_from_hist(hd_ref)
    y = jnp.dot(agg, w_ref[...], preferred_element_type=jnp.float32)
    o_ref[...] = y + b_ref[...][None, :]


_f32rows = jax.ShapeDtypeStruct((N_ROWS, D), jnp.float32)

_tc_scale0 = pl.pallas_call(_tc_scale0_body, out_shape=_f32rows)
_tc_layer1 = pl.pallas_call(_tc_layer1_body, out_shape=_f32rows)
_tc_layer2 = pl.pallas_call(_tc_layer2_body, out_shape=_f32rows)


@jax.jit
def kernel(node_ids, edge_index, emb, W1, b1, W2, b2):
    src = edge_index[0].astype(jnp.int32)
    dst = edge_index[1].astype(jnp.int32)
    epad = jnp.full((E_PAD - N_EDGES,), PAD, jnp.int32)
    src_p = jnp.concatenate([src, epad])
    dst_p = jnp.concatenate([dst, epad])
    ids_p = jnp.concatenate(
        [node_ids.astype(jnp.int32),
         jnp.zeros((IDS_PAD - N_NODES,), jnp.int32)])

    h0, hsrc, hdst = _sc_prep(ids_p, src_p, dst_p, emb)
    h0s = _tc_scale0(h0, hsrc)
    parts1 = _sc_agg(h0s, src_p, dst_p)
    h1s = _tc_layer1(parts1, hdst, hsrc, W1, b1)
    parts2 = _sc_agg(h1s, src_p, dst_p)
    out = _tc_layer2(parts2, hdst, W2, b2)
    return out[:N_NODES]


# speedup vs baseline: 2.1150x; 2.1150x over previous
import functools

import jax
import jax.numpy as jnp
from jax import lax
from jax.experimental import pallas as pl
from jax.experimental.pallas import tpu as pltpu
from jax.experimental.pallas import tpu_sc as plsc

N_NODES = 10000
D = 128
NC = 2
NS = 16
L = 16
NW = NC * NS
CHUNK = 128
N_ROWS = 10240
PAD = N_NODES
ROWS_PER_SUB = N_ROWS // NS
N_EDGES = 320000
E_PAD = ((N_EDGES + NW * CHUNK - 1) // (NW * CHUNK)) * (NW * CHUNK)
E_CHUNKS = E_PAD // CHUNK
CPT = E_CHUNKS // NW
IDS_PAD = 12288
IDS_CHUNKS = IDS_PAD // CHUNK
IDS_CPT = IDS_CHUNKS // NW

_mesh = plsc.VectorSubcoreMesh(core_axis_name="c", subcore_axis_name="s")


@functools.partial(
    pl.kernel,
    out_type=[
        jax.ShapeDtypeStruct((NC, N_ROWS, D), jnp.float32),
        jax.ShapeDtypeStruct((IDS_PAD, D), jnp.float32),
    ],
    mesh=_mesh,
    scratch_types=[
        pltpu.VMEM((1, CHUNK), jnp.int32),
        pltpu.VMEM((1, CHUNK), jnp.int32),
        pltpu.VMEM((CHUNK, D), jnp.float32),
        pltpu.VMEM((CHUNK, D), jnp.float32),
        pltpu.VMEM_SHARED((N_ROWS, D), jnp.float32),
    ],
)
def _sc_pass(table_hbm, gat_hbm, scat_hbm, ids_hbm, emb_hbm,
             parts_hbm, h0_hbm,
             gidx_v, sidx_v, rows_v, zrow_v, agg_sh):
    c = lax.axis_index("c")
    s = lax.axis_index("s")
    wid = s * NC + c
    sub_rows = pl.ds(s * ROWS_PER_SUB, ROWS_PER_SUB)

    @pl.loop(0, CHUNK)
    def _(i):
        @pl.loop(0, D // L)
        def _(k):
            zrow_v[i, pl.ds(k * L, L)] = jnp.zeros((L,), jnp.float32)

    @pl.loop(0, ROWS_PER_SUB // CHUNK)
    def _(r):
        pltpu.sync_copy(
            zrow_v, agg_sh.at[pl.ds(s * ROWS_PER_SUB + r * CHUNK, CHUNK)])
    plsc.subcore_barrier()

    @pl.loop(0, CPT)
    def _(j):
        chunk = wid * CPT + j
        pltpu.sync_copy(gat_hbm.at[pl.ds(chunk, 1)], gidx_v)
        pltpu.sync_copy(scat_hbm.at[pl.ds(chunk, 1)], sidx_v)
        pltpu.sync_copy(table_hbm.at[gidx_v.at[0]], rows_v)
        pltpu.sync_copy(rows_v, agg_sh.at[sidx_v.at[0]], add=True)

    @pl.loop(0, IDS_CPT)
    def _(j):
        chunk = wid * IDS_CPT + j
        pltpu.sync_copy(ids_hbm.at[pl.ds(chunk, 1)], gidx_v)
        pltpu.sync_copy(emb_hbm.at[gidx_v.at[0]], rows_v)
        pltpu.sync_copy(rows_v, h0_hbm.at[pl.ds(chunk * CHUNK, CHUNK)])

    plsc.subcore_barrier()
    pltpu.sync_copy(agg_sh.at[sub_rows], parts_hbm.at[c, sub_rows])


def _norm_from_parts(p_ref):
    deg = p_ref[0, :, 0:1] + p_ref[1, :, 0:1]
    return lax.rsqrt(jnp.maximum(deg, 1.0))


def _tc_scale0_body(h0_ref, ds_ref, o_ref):
    o_ref[...] = h0_ref[:N_ROWS, :] * _norm_from_parts(ds_ref)


def _tc_layer1_body(p_ref, dd_ref, ds_ref, w_ref, b_ref, o_ref):
    agg = (p_ref[0] + p_ref[1]) * _norm_from_parts(dd_ref)
    y = jnp.dot(agg, w_ref[...], preferred_element_type=jnp.float32)
    y = y + b_ref[...][None, :]
    o_ref[...] = jnp.maximum(y, 0.0) * _norm_from_parts(ds_ref)


def _tc_layer2_body(p_ref, dd_ref, w_ref, b_ref, o_ref):
    agg = (p_ref[0] + p_ref[1]) * _norm_from_parts(dd_ref)
    y = jnp.dot(agg, w_ref[...], preferred_element_type=jnp.float32)
    o_ref[...] = y + b_ref[...][None, :]


_f32rows = jax.ShapeDtypeStruct((N_ROWS, D), jnp.float32)

_tc_scale0 = pl.pallas_call(_tc_scale0_body, out_shape=_f32rows)
_tc_layer1 = pl.pallas_call(_tc_layer1_body, out_shape=_f32rows)
_tc_layer2 = pl.pallas_call(_tc_layer2_body, out_shape=_f32rows)


@jax.jit
def kernel(node_ids, edge_index, emb, W1, b1, W2, b2):
    src = edge_index[0].astype(jnp.int32)
    dst = edge_index[1].astype(jnp.int32)
    epad = jnp.full((E_PAD - N_EDGES,), PAD, jnp.int32)
    src_p = jnp.concatenate([src, epad]).reshape(E_CHUNKS, CHUNK)
    dst_p = jnp.concatenate([dst, epad]).reshape(E_CHUNKS, CHUNK)
    ids_p = jnp.concatenate(
        [node_ids.astype(jnp.int32),
         jnp.zeros((IDS_PAD - N_NODES,), jnp.int32)]).reshape(IDS_CHUNKS, CHUNK)
    ones_tab = jnp.ones((N_ROWS, D), jnp.float32)

    deg_s_parts, h0 = _sc_pass(ones_tab, src_p, src_p, ids_p, emb)
    deg_d_parts, _ = _sc_pass(ones_tab, dst_p, dst_p, ids_p, emb)
    h0s = _tc_scale0(h0, deg_s_parts)
    parts1, _ = _sc_pass(h0s, src_p, dst_p, ids_p, emb)
    h1s = _tc_layer1(parts1, deg_d_parts, deg_s_parts, W1, b1)
    parts2, _ = _sc_pass(h1s, src_p, dst_p, ids_p, emb)
    out = _tc_layer2(parts2, deg_d_parts, W2, b2)
    return out[:N_NODES]


# speedup vs baseline: 3.8537x; 1.8221x over previous
import dataclasses
import functools

import jax
import jax.numpy as jnp
from jax import lax
from jax.experimental import pallas as pl
from jax.experimental.pallas import tpu as pltpu
from jax.experimental.pallas import tpu_sc as plsc

N_NODES = 10000
D = 128
NC = 2
NS = 16
L = 16
NW = NC * NS
CHUNK = 128
N_ROWS = 10112
PAD = N_NODES
ROWS_PER_SUB = N_ROWS // NS
N_EDGES = 320000
E_PAD = ((N_EDGES + NW * CHUNK - 1) // (NW * CHUNK)) * (NW * CHUNK)
E_CHUNKS = E_PAD // CHUNK
CPT = E_CHUNKS // NW
IDS_PAD = 12288
IDS_CHUNKS = IDS_PAD // CHUNK
IDS_CPT = IDS_CHUNKS // NW

_mesh = plsc.VectorSubcoreMesh(core_axis_name="c", subcore_axis_name="s")

_cp = pltpu.CompilerParams()
if "needs_layout_passes" in pltpu.CompilerParams.__dataclass_fields__:
    _cp = dataclasses.replace(_cp, needs_layout_passes=False)


@functools.partial(
    pl.kernel,
    out_type=[
        jax.ShapeDtypeStruct((NC, N_ROWS, D), jnp.float32),
        jax.ShapeDtypeStruct((IDS_PAD, D), jnp.float32),
    ],
    mesh=_mesh,
    compiler_params=_cp,
    scratch_types=[
        pltpu.VMEM((1, CHUNK), jnp.int32),
        pltpu.VMEM((1, CHUNK), jnp.int32),
        pltpu.VMEM((1, CHUNK), jnp.int32),
        pltpu.VMEM((1, CHUNK), jnp.int32),
        pltpu.VMEM((CHUNK, D), jnp.float32),
        pltpu.VMEM((CHUNK, D), jnp.float32),
        pltpu.VMEM((CHUNK, D), jnp.float32),
        pltpu.VMEM_SHARED((N_ROWS, D), jnp.float32),
        pltpu.VMEM((L,), jnp.int32),
    ],
)
def _sc_pass(table_hbm, gat_hbm, scat_hbm, ids_hbm, emb_hbm, mode_hbm,
             parts_hbm, h0_hbm,
             gidx_a, sidx_a, gidx_b, sidx_b, rows_a, rows_b, zrow_v, agg_sh,
             mode_v):
    c = lax.axis_index("c")
    s = lax.axis_index("s")
    wid = s * NC + c
    sub_rows = pl.ds(s * ROWS_PER_SUB, ROWS_PER_SUB)

    @pl.loop(0, CHUNK)
    def _(i):
        @pl.loop(0, D // L)
        def _(k):
            zrow_v[i, pl.ds(k * L, L)] = jnp.zeros((L,), jnp.float32)

    @pl.loop(0, ROWS_PER_SUB // CHUNK)
    def _(r):
        pltpu.sync_copy(
            zrow_v, agg_sh.at[pl.ds(s * ROWS_PER_SUB + r * CHUNK, CHUNK)])

    _REM = ROWS_PER_SUB % CHUNK
    if _REM:
        pltpu.sync_copy(
            zrow_v.at[pl.ds(0, _REM)],
            agg_sh.at[pl.ds(s * ROWS_PER_SUB + (ROWS_PER_SUB // CHUNK) * CHUNK,
                            _REM)])
    pltpu.sync_copy(mode_hbm, mode_v)
    plsc.subcore_barrier()
    base = wid * CPT
    mode = jnp.max(mode_v[...])

    def load_gidx(j, gidx):
        pltpu.sync_copy(gat_hbm.at[pl.ds(base + j, 1)], gidx)

    def load_sidx(j, sidx):
        pltpu.sync_copy(scat_hbm.at[pl.ds(base + j, 1)], sidx)

    def gather(gidx, rows, sem):
        return pltpu.make_async_copy(table_hbm.at[gidx.at[0]], rows, sem)

    def scat(sidx, rows, sem):
        return pltpu.make_async_copy(rows, agg_sh.at[sidx.at[0]], sem)

    def pipeline(gsem_a, gsem_b, ssem_a, ssem_b):
        @pl.when((mode & 1) == 0)
        def _():
            load_gidx(0, gidx_a)
            load_sidx(0, sidx_a)
            gather(gidx_a, rows_a, gsem_a).start()

            @pl.loop(0, (CPT - 1) // 2)
            def _(t):
                ja = 2 * t
                jb = 2 * t + 1
                gather(gidx_a, rows_a, gsem_a).wait()
                scat(sidx_a, rows_a, ssem_a).start(add=True)

                @pl.when(t > 0)
                def _():
                    scat(sidx_b, rows_b, ssem_b).wait()
                load_gidx(jb, gidx_b)
                load_sidx(jb, sidx_b)
                gather(gidx_b, rows_b, gsem_b).start()
                gather(gidx_b, rows_b, gsem_b).wait()
                scat(sidx_b, rows_b, ssem_b).start(add=True)
                scat(sidx_a, rows_a, ssem_a).wait()
                load_gidx(ja + 2, gidx_a)
                load_sidx(ja + 2, sidx_a)
                gather(gidx_a, rows_a, gsem_a).start()

            gather(gidx_a, rows_a, gsem_a).wait()
            scat(sidx_a, rows_a, ssem_a).start(add=True)
            scat(sidx_b, rows_b, ssem_b).wait()
            scat(sidx_a, rows_a, ssem_a).wait()

        @pl.when((mode & 1) == 1)
        def _():
            @pl.loop(0, CHUNK)
            def _(i):
                @pl.loop(0, D // L)
                def _(k):
                    zrow_v[i, pl.ds(k * L, L)] = jnp.full((L,), 1.0,
                                                          jnp.float32)

            load_sidx(0, sidx_a)
            scat(sidx_a, zrow_v, ssem_a).start(add=True)
            load_sidx(1, sidx_b)
            scat(sidx_b, zrow_v, ssem_b).start(add=True)

            @pl.loop(0, (CPT - 3) // 2 + 1)
            def _(t):
                scat(sidx_a, zrow_v, ssem_a).wait()
                load_sidx(2 * t + 2, sidx_a)
                scat(sidx_a, zrow_v, ssem_a).start(add=True)

                @pl.when(2 * t + 3 < CPT)
                def _():
                    scat(sidx_b, zrow_v, ssem_b).wait()
                    load_sidx(2 * t + 3, sidx_b)
                    scat(sidx_b, zrow_v, ssem_b).start(add=True)

            scat(sidx_b, zrow_v, ssem_b).wait()
            scat(sidx_a, zrow_v, ssem_a).wait()

    pl.run_scoped(pipeline, pltpu.SemaphoreType.DMA, pltpu.SemaphoreType.DMA,
                  pltpu.SemaphoreType.DMA, pltpu.SemaphoreType.DMA)

    @pl.when((mode & 2) != 0)
    def _():
        @pl.loop(0, IDS_CPT)
        def _(j):
            chunk = wid * IDS_CPT + j
            pltpu.sync_copy(ids_hbm.at[pl.ds(chunk, 1)], gidx_a)
            pltpu.sync_copy(emb_hbm.at[gidx_a.at[0]], rows_a)
            pltpu.sync_copy(rows_a, h0_hbm.at[pl.ds(chunk * CHUNK, CHUNK)])

    plsc.subcore_barrier()
    pltpu.sync_copy(agg_sh.at[sub_rows], parts_hbm.at[c, sub_rows])


def _norm_from_parts(p_ref):
    deg = p_ref[0, :, 0:1] + p_ref[1, :, 0:1]
    return lax.rsqrt(jnp.maximum(deg, 1.0))


def _tc_scale0_body(h0_ref, ds_ref, o_ref):
    o_ref[...] = h0_ref[:N_ROWS, :] * _norm_from_parts(ds_ref)


def _tc_layer1_body(p_ref, dd_ref, ds_ref, w_ref, b_ref, o_ref):
    agg = (p_ref[0] + p_ref[1]) * _norm_from_parts(dd_ref)
    y = jnp.dot(agg, w_ref[...], preferred_element_type=jnp.float32)
    y = y + b_ref[...][None, :]
    o_ref[...] = jnp.maximum(y, 0.0) * _norm_from_parts(ds_ref)


def _tc_layer2_body(p_ref, dd_ref, w_ref, b_ref, o_ref):
    agg = (p_ref[0] + p_ref[1]) * _norm_from_parts(dd_ref)
    y = jnp.dot(agg, w_ref[...], preferred_element_type=jnp.float32)
    o_ref[...] = y + b_ref[...][None, :]


_f32rows = jax.ShapeDtypeStruct((N_ROWS, D), jnp.float32)

_tc_scale0 = pl.pallas_call(_tc_scale0_body, out_shape=_f32rows)
_tc_layer1 = pl.pallas_call(_tc_layer1_body, out_shape=_f32rows)
_tc_layer2 = pl.pallas_call(_tc_layer2_body, out_shape=_f32rows)


@jax.jit
def kernel(node_ids, edge_index, emb, W1, b1, W2, b2):
    src = edge_index[0].astype(jnp.int32)
    dst = edge_index[1].astype(jnp.int32)
    epad = jnp.full((E_PAD - N_EDGES,), PAD, jnp.int32)
    src_p = jnp.concatenate([src, epad]).reshape(E_CHUNKS, CHUNK)
    dst_p = jnp.concatenate([dst, epad]).reshape(E_CHUNKS, CHUNK)
    ids_p = jnp.concatenate(
        [node_ids.astype(jnp.int32),
         jnp.zeros((IDS_PAD - N_NODES,), jnp.int32)]).reshape(IDS_CHUNKS, CHUNK)
    ones_tab = jnp.ones((N_ROWS, D), jnp.float32)

    m_deg_emb = jnp.full((L,), 3, jnp.int32)
    m_deg = jnp.full((L,), 1, jnp.int32)
    m_agg = jnp.zeros((L,), jnp.int32)
    deg_s_parts, h0 = _sc_pass(ones_tab, src_p, src_p, ids_p, emb, m_deg_emb)
    deg_d_parts, _ = _sc_pass(ones_tab, dst_p, dst_p, ids_p, emb, m_deg)
    h0s = _tc_scale0(h0, deg_s_parts)
    parts1, _ = _sc_pass(h0s, src_p, dst_p, ids_p, emb, m_agg)
    h1s = _tc_layer1(parts1, deg_d_parts, deg_s_parts, W1, b1)
    parts2, _ = _sc_pass(h1s, src_p, dst_p, ids_p, emb, m_agg)
    out = _tc_layer2(parts2, deg_d_parts, W2, b2)
    return out[:N_NODES]
